# baseline (speedup 1.0000x reference)
import jax
import jax.numpy as jnp
import numpy as np
from jax.experimental import pallas as pl
from jax.experimental.pallas import tpu as pltpu

_K = 512


def _topk_mask_kernel(x_ref, o_ref, key_ref):
    x = x_ref[...]
    key_ref[...] = jax.lax.bitcast_convert_type(jnp.maximum(x, 0.0), jnp.int32)
    rows, n = x.shape
    n_chunks = n // 128

    def body(i, prefix):
        shift = 30 - i
        cand = prefix + jnp.left_shift(np.int32(1), shift)
        accs = []
        for g in range(16):
            acc = None
            for j in range(n_chunks // 16):
                c = g * (n_chunks // 16) + j
                s = key_ref[:, c * 128:(c + 1) * 128]
                t = jnp.where(s >= cand, 1.0, 0.0)
                acc = t if acc is None else acc + t
            accs.append(acc)
        while len(accs) > 1:
            accs = [accs[k] + accs[k + 1] for k in range(0, len(accs), 2)]
        cnt = jnp.sum(accs[0], axis=1, keepdims=True)
        return jnp.where(cnt >= _K, cand, prefix)

    thresh = jax.lax.fori_loop(
        0, 31, body, jnp.zeros((rows, 1), jnp.int32), unroll=31
    )
    key = key_ref[...]
    y = jax.lax.bitcast_convert_type(key, jnp.float32)
    o_ref[...] = jnp.where(key >= thresh, y, 0.0)


def kernel(x):
    m, n = x.shape
    block_rows = 64
    return pl.pallas_call(
        _topk_mask_kernel,
        grid=(m // block_rows,),
        in_specs=[pl.BlockSpec((block_rows, n), lambda i: (i, 0))],
        out_specs=pl.BlockSpec((block_rows, n), lambda i: (i, 0)),
        out_shape=jax.ShapeDtypeStruct((m, n), x.dtype),
        scratch_shapes=[pltpu.VMEM((block_rows, n), jnp.int32)],
    )(x)


# speedup vs baseline: 1.0357x; 1.0357x over previous
import jax
import jax.numpy as jnp
import numpy as np
from jax.experimental import pallas as pl
from jax.experimental.pallas import tpu as pltpu

_K = 512


def _topk_mask_kernel(x_ref, o_ref, key_ref):
    x = x_ref[...]
    key_ref[...] = jax.lax.bitcast_convert_type(jnp.maximum(x, 0.0), jnp.int32)
    rows, n = x.shape
    n_chunks = n // 128

    def body(i, prefix):
        shift = 30 - i
        cand = prefix + jnp.left_shift(np.int32(1), shift)
        accs = []
        for g in range(16):
            acc = None
            for j in range(n_chunks // 16):
                c = g * (n_chunks // 16) + j
                s = key_ref[:, c * 128:(c + 1) * 128]
                t = jnp.where(s >= cand, 1.0, 0.0)
                acc = t if acc is None else acc + t
            accs.append(acc)
        while len(accs) > 1:
            accs = [accs[k] + accs[k + 1] for k in range(0, len(accs), 2)]
        cnt = jnp.sum(accs[0], axis=1, keepdims=True)
        return jnp.where(cnt >= _K, cand, prefix)

    thresh = jax.lax.fori_loop(
        0, 31, body, jnp.zeros((rows, 1), jnp.int32), unroll=16
    )
    key = key_ref[...]
    y = jax.lax.bitcast_convert_type(key, jnp.float32)
    o_ref[...] = jnp.where(key >= thresh, y, 0.0)


def kernel(x):
    m, n = x.shape
    block_rows = 32
    return pl.pallas_call(
        _topk_mask_kernel,
        grid=(m // block_rows,),
        in_specs=[pl.BlockSpec((block_rows, n), lambda i: (i, 0))],
        out_specs=pl.BlockSpec((block_rows, n), lambda i: (i, 0)),
        out_shape=jax.ShapeDtypeStruct((m, n), x.dtype),
        scratch_shapes=[pltpu.VMEM((block_rows, n), jnp.int32)],
    )(x)
